# baseline (speedup 1.0000x reference)
import functools

import jax
import jax.numpy as jnp
from jax import lax
from jax.experimental import pallas as pl
from jax.experimental.pallas import tpu as pltpu
from jax.experimental.pallas import tpu_sc as plsc

BATCH = 16384
EMBED_DIM = 64
NC = 2
NS = 16
NW = NC * NS
B_PER_W = BATCH // NW
GATHER_CHUNK = 128
N_CHUNKS = B_PER_W // GATHER_CHUNK
GROUP = 16
N_GROUPS = B_PER_W // GROUP
K = EMBED_DIM // 16


def _body(uid_hbm, aid_hbm, ut_hbm, at_hbm, out_hbm,
          uidx_v, aidx_v, urows_v, arows_v, scr_v, out_v, sem):
    wid = lax.axis_index("s") * NC + lax.axis_index("c")
    base = wid * B_PER_W

    pltpu.sync_copy(uid_hbm.at[pl.ds(base, B_PER_W)], uidx_v)
    pltpu.sync_copy(aid_hbm.at[pl.ds(base, B_PER_W)], aidx_v)

    copies = []
    for j in range(N_CHUNKS):
        sl = pl.ds(j * GATHER_CHUNK, GATHER_CHUNK)
        copies.append(pltpu.async_copy(ut_hbm.at[uidx_v.at[sl]], urows_v.at[sl], sem))
        copies.append(pltpu.async_copy(at_hbm.at[aidx_v.at[sl]], arows_v.at[sl], sem))
    for c in copies:
        c.wait()

    lane = lax.iota(jnp.int32, 16)
    col_base = lane * GROUP

    def group_step(g, carry):
        row0 = g * GROUP
        for r in range(GROUP):
            row = row0 + r
            acc = urows_v[row, pl.ds(0, 16)] * arows_v[row, pl.ds(0, 16)]
            for k in range(1, K):
                acc = acc + urows_v[row, pl.ds(k * 16, 16)] * arows_v[row, pl.ds(k * 16, 16)]
            scr_v[pl.ds(r * 16, 16)] = acc
        tot = plsc.load_gather(scr_v, [col_base])
        for c in range(1, 16):
            tot = tot + plsc.load_gather(scr_v, [col_base + c])
        out_v[pl.ds(row0, 16)] = 1.0 / (1.0 + jnp.exp(-tot))
        return carry

    lax.fori_loop(0, N_GROUPS, group_step, 0)

    pltpu.sync_copy(out_v, out_hbm.at[pl.ds(base, B_PER_W)])


@jax.jit
def _run(user_ids, anime_ids, user_table, anime_table):
    mesh = plsc.VectorSubcoreMesh(core_axis_name="c", subcore_axis_name="s")
    k = functools.partial(
        pl.kernel,
        mesh=mesh,
        compiler_params=pltpu.CompilerParams(
            needs_layout_passes=False, use_tc_tiling_on_sc=False),
        out_type=jax.ShapeDtypeStruct((BATCH,), jnp.float32),
        scratch_types=[
            pltpu.VMEM((B_PER_W,), jnp.int32),
            pltpu.VMEM((B_PER_W,), jnp.int32),
            pltpu.VMEM((B_PER_W, EMBED_DIM), jnp.float32),
            pltpu.VMEM((B_PER_W, EMBED_DIM), jnp.float32),
            pltpu.VMEM((GROUP * 16,), jnp.float32),
            pltpu.VMEM((B_PER_W,), jnp.float32),
            pltpu.SemaphoreType.DMA,
        ],
    )(_body)
    return k(user_ids, anime_ids, user_table, anime_table)


def kernel(user_ids, anime_ids, user_table, anime_table):
    return _run(jnp.asarray(user_ids, jnp.int32), jnp.asarray(anime_ids, jnp.int32),
                user_table, anime_table)
